# Initial kernel scaffold; baseline (speedup 1.0000x reference)
import jax
import jax.numpy as jnp
from jax.experimental import pallas as pl


def kernel(fusion):
    raise NotImplementedError("write your pallas kernel here")


# speedup vs baseline: 1814.2471x; 1814.2471x over previous
import functools

import jax
import jax.numpy as jnp
from jax import lax
from jax.experimental import pallas as pl
from jax.experimental.pallas import tpu as pltpu
from jax.experimental.pallas import tpu_sc as plsc

LEVEL = 255
NC, NS, L = 2, 16, 16
NW = NC * NS
B, C, H, W = 16, 3, 512, 512
NPIX = C * H * W
TOTAL = B * NPIX
PER_W = TOTAL // NW
CH = 8192
NCHUNK = PER_W // CH
VPC = CH // L
UNROLL = 8
HBINS = 256

_mesh = plsc.VectorSubcoreMesh(
    core_axis_name="c", subcore_axis_name="s", num_cores=NC, num_subcores=NS
)
_params = pltpu.CompilerParams(needs_layout_passes=False)


def _bin_of(x):
    x = jnp.minimum(jnp.maximum(x, 0.0), 1.0)
    y = x * 255.0
    b = (LEVEL - 1) - (255.0 - y).astype(jnp.int32)
    return jnp.minimum(jnp.maximum(b, 0), LEVEL - 1)


@functools.partial(
    pl.kernel,
    out_type=jax.ShapeDtypeStruct((NW * HBINS,), jnp.float32),
    mesh=_mesh,
    scratch_types=[
        pltpu.VMEM((CH,), jnp.float32),
        pltpu.VMEM((L * HBINS,), jnp.float32),
        pltpu.VMEM((HBINS,), jnp.float32),
    ],
    compiler_params=_params,
)
def _hist_kernel(x_hbm, part_hbm, buf_v, hist_v, row_v):
    wid = lax.axis_index("c") * NS + lax.axis_index("s")
    base = wid * PER_W
    lane = lax.iota(jnp.int32, L)
    ones = jnp.full((L,), 1.0, dtype=jnp.float32)

    def zero_body(i, _):
        hist_v[pl.ds(i * L, L)] = jnp.zeros((L,), jnp.float32)
        return 0
    lax.fori_loop(0, (L * HBINS) // L, zero_body, 0)

    def chunk_body(ci, _):
        pltpu.sync_copy(x_hbm.at[pl.ds(base + ci * CH, CH)], buf_v)

        def vreg_body(vi, _):
            for u in range(UNROLL):
                j = (vi * UNROLL + u) * L
                bins = _bin_of(buf_v[pl.ds(j, L)])
                idx = lane * HBINS + bins
                plsc.addupdate_scatter(hist_v, [idx], ones)
            return 0
        lax.fori_loop(0, VPC // UNROLL, vreg_body, 0)
        return 0
    lax.fori_loop(0, NCHUNK, chunk_body, 0)

    for k in range(HBINS // L):
        acc = hist_v[pl.ds(k * L, L)]
        for l in range(1, L):
            acc = acc + hist_v[pl.ds(l * HBINS + k * L, L)]
        row_v[pl.ds(k * L, L)] = acc

    pltpu.sync_copy(row_v, part_hbm.at[pl.ds(wid * HBINS, HBINS)])


@functools.partial(
    pl.kernel,
    out_type=jax.ShapeDtypeStruct((TOTAL,), jnp.float32),
    mesh=_mesh,
    scratch_types=[
        pltpu.VMEM((CH,), jnp.float32),
        pltpu.VMEM((CH,), jnp.float32),
        pltpu.VMEM((HBINS,), jnp.float32),
        pltpu.VMEM((HBINS,), jnp.float32),
        pltpu.VMEM((HBINS,), jnp.float32),
    ],
    compiler_params=_params,
)
def _map_kernel(x_hbm, part_hbm, out_hbm, buf_v, obuf_v, p0_v, p1_v, cdf_v):
    wid = lax.axis_index("c") * NS + lax.axis_index("s")
    img = wid // 2
    base = wid * PER_W

    pltpu.sync_copy(part_hbm.at[pl.ds((2 * img) * HBINS, HBINS)], p0_v)
    pltpu.sync_copy(part_hbm.at[pl.ds((2 * img + 1) * HBINS, HBINS)], p1_v)

    inv_n = jnp.float32(1.0 / NPIX)
    running = jnp.float32(0.0)
    for k in range(HBINS // L):
        v = p0_v[pl.ds(k * L, L)] + p1_v[pl.ds(k * L, L)]
        c = plsc.cumsum(v) + running
        cdf_v[pl.ds(k * L, L)] = c * inv_n
        running = running + jnp.sum(v)

    def chunk_body(ci, _):
        off = base + ci * CH
        pltpu.sync_copy(x_hbm.at[pl.ds(off, CH)], buf_v)

        def vreg_body(vi, _):
            for u in range(UNROLL):
                j = (vi * UNROLL + u) * L
                bins = _bin_of(buf_v[pl.ds(j, L)])
                obuf_v[pl.ds(j, L)] = plsc.load_gather(cdf_v, [bins])
            return 0
        lax.fori_loop(0, VPC // UNROLL, vreg_body, 0)

        pltpu.sync_copy(obuf_v, out_hbm.at[pl.ds(off, CH)])
        return 0
    lax.fori_loop(0, NCHUNK, chunk_body, 0)


@jax.jit
def kernel(fusion):
    x = fusion.reshape(TOTAL)
    partials = _hist_kernel(x)
    out = _map_kernel(x, partials)
    return out.reshape(fusion.shape)


# speedup vs baseline: 2310.3482x; 1.2734x over previous
import functools

import jax
import jax.numpy as jnp
from jax import lax
from jax.experimental import pallas as pl
from jax.experimental.pallas import tpu as pltpu
from jax.experimental.pallas import tpu_sc as plsc

LEVEL = 255
NC, NS, L = 2, 16, 16
NW = NC * NS
B, C, H, W = 16, 3, 512, 512
NPIX = C * H * W
TOTAL = B * NPIX
PER_W = TOTAL // NW
CH = 16384
NCHUNK = PER_W // CH
VPC = CH // L
UNROLL = 16
HBINS = 256

_mesh = plsc.VectorSubcoreMesh(
    core_axis_name="c", subcore_axis_name="s", num_cores=NC, num_subcores=NS
)
_params = pltpu.CompilerParams(needs_layout_passes=False)


def _bin_of(x):
    t = (255.0 - x * 255.0).astype(jnp.int32)
    return jnp.maximum((LEVEL - 1) - t, 0)


@functools.partial(
    pl.kernel,
    out_type=jax.ShapeDtypeStruct((NW * HBINS,), jnp.float32),
    mesh=_mesh,
    scratch_types=[
        pltpu.VMEM((CH,), jnp.float32),
        pltpu.VMEM((CH,), jnp.float32),
        pltpu.VMEM((L * HBINS,), jnp.float32),
        pltpu.VMEM((HBINS,), jnp.float32),
        pltpu.SemaphoreType.DMA,
        pltpu.SemaphoreType.DMA,
    ],
    compiler_params=_params,
)
def _hist_kernel(x_hbm, part_hbm, buf0_v, buf1_v, hist_v, row_v, sem0, sem1):
    wid = lax.axis_index("c") * NS + lax.axis_index("s")
    base = wid * PER_W
    bufs, sems = (buf0_v, buf1_v), (sem0, sem1)
    lane_off = lax.iota(jnp.int32, L) * HBINS
    ones = jnp.full((L,), 1.0, dtype=jnp.float32)

    def start_in(ci, b):
        pltpu.async_copy(x_hbm.at[pl.ds(base + ci * CH, CH)], bufs[b], sems[b])

    def wait_in(b):
        pltpu.make_async_copy(
            x_hbm.at[pl.ds(base, CH)], bufs[b], sems[b]
        ).wait()

    start_in(0, 0)
    start_in(1, 1)

    def zero_body(i, _):
        hist_v[pl.ds(i * L, L)] = jnp.zeros((L,), jnp.float32)
        return 0
    lax.fori_loop(0, (L * HBINS) // L, zero_body, 0)

    def chunk_pair(g, _):
        for b in range(2):
            ci = g * 2 + b
            wait_in(b)

            def vreg_body(vi, _, buf=bufs[b]):
                for u in range(UNROLL):
                    j = (vi * UNROLL + u) * L
                    idx = lane_off + _bin_of(buf[pl.ds(j, L)])
                    plsc.addupdate_scatter(hist_v, [idx], ones)
                return 0
            lax.fori_loop(0, VPC // UNROLL, vreg_body, 0)

            @pl.when(ci + 2 < NCHUNK)
            def _():
                start_in(ci + 2, b)
        return 0
    lax.fori_loop(0, NCHUNK // 2, chunk_pair, 0)

    for k in range(HBINS // L):
        acc = hist_v[pl.ds(k * L, L)]
        for l in range(1, L):
            acc = acc + hist_v[pl.ds(l * HBINS + k * L, L)]
        row_v[pl.ds(k * L, L)] = acc

    pltpu.sync_copy(row_v, part_hbm.at[pl.ds(wid * HBINS, HBINS)])


@functools.partial(
    pl.kernel,
    out_type=jax.ShapeDtypeStruct((TOTAL,), jnp.float32),
    mesh=_mesh,
    scratch_types=[
        pltpu.VMEM((CH,), jnp.float32),
        pltpu.VMEM((CH,), jnp.float32),
        pltpu.VMEM((CH,), jnp.float32),
        pltpu.VMEM((CH,), jnp.float32),
        pltpu.VMEM((HBINS,), jnp.float32),
        pltpu.VMEM((HBINS,), jnp.float32),
        pltpu.VMEM((HBINS,), jnp.float32),
        pltpu.SemaphoreType.DMA,
        pltpu.SemaphoreType.DMA,
        pltpu.SemaphoreType.DMA,
        pltpu.SemaphoreType.DMA,
    ],
    compiler_params=_params,
)
def _map_kernel(x_hbm, part_hbm, out_hbm, buf0_v, buf1_v, obuf0_v, obuf1_v,
                p0_v, p1_v, cdf_v, isem0, isem1, osem0, osem1):
    wid = lax.axis_index("c") * NS + lax.axis_index("s")
    img = wid // 2
    base = wid * PER_W
    bufs, isems = (buf0_v, buf1_v), (isem0, isem1)
    obufs, osems = (obuf0_v, obuf1_v), (osem0, osem1)

    def start_in(ci, b):
        pltpu.async_copy(x_hbm.at[pl.ds(base + ci * CH, CH)], bufs[b], isems[b])

    def wait_in(b):
        pltpu.make_async_copy(
            x_hbm.at[pl.ds(base, CH)], bufs[b], isems[b]
        ).wait()

    def start_out(ci, b):
        pltpu.async_copy(
            obufs[b], out_hbm.at[pl.ds(base + ci * CH, CH)], osems[b]
        )

    def wait_out(b):
        pltpu.make_async_copy(
            obufs[b], out_hbm.at[pl.ds(base, CH)], osems[b]
        ).wait()

    start_in(0, 0)
    start_in(1, 1)

    pltpu.sync_copy(part_hbm.at[pl.ds((2 * img) * HBINS, HBINS)], p0_v)
    pltpu.sync_copy(part_hbm.at[pl.ds((2 * img + 1) * HBINS, HBINS)], p1_v)

    inv_n = jnp.float32(1.0 / NPIX)
    running = jnp.zeros((L,), jnp.float32)
    for k in range(HBINS // L):
        v = p0_v[pl.ds(k * L, L)] + p1_v[pl.ds(k * L, L)]
        c = plsc.cumsum(v) + running
        cdf_v[pl.ds(k * L, L)] = c * inv_n
        running = running + jnp.sum(v)

    def chunk_pair(g, _):
        for b in range(2):
            ci = g * 2 + b
            wait_in(b)

            @pl.when(ci >= 2)
            def _():
                wait_out(b)

            def vreg_body(vi, _, buf=bufs[b], obuf=obufs[b]):
                for u in range(UNROLL):
                    j = (vi * UNROLL + u) * L
                    bins = _bin_of(buf[pl.ds(j, L)])
                    obuf[pl.ds(j, L)] = plsc.load_gather(cdf_v, [bins])
                return 0
            lax.fori_loop(0, VPC // UNROLL, vreg_body, 0)

            start_out(ci, b)

            @pl.when(ci + 2 < NCHUNK)
            def _():
                start_in(ci + 2, b)
        return 0
    lax.fori_loop(0, NCHUNK // 2, chunk_pair, 0)

    wait_out(0)
    wait_out(1)


@jax.jit
def kernel(fusion):
    x = fusion.reshape(TOTAL)
    partials = _hist_kernel(x)
    out = _map_kernel(x, partials)
    return out.reshape(fusion.shape)


# speedup vs baseline: 2344.9842x; 1.0150x over previous
import functools

import jax
import jax.numpy as jnp
from jax import lax
from jax.experimental import pallas as pl
from jax.experimental.pallas import tpu as pltpu
from jax.experimental.pallas import tpu_sc as plsc

LEVEL = 255
NC, NS, L = 2, 16, 16
NW = NC * NS
B, C, H, W = 16, 3, 512, 512
NPIX = C * H * W
TOTAL = B * NPIX
PER_W = TOTAL // NW
CH = 16384
NCHUNK = PER_W // CH
VPC = CH // L
UNROLL = 16
HBINS = 256

_mesh = plsc.VectorSubcoreMesh(
    core_axis_name="c", subcore_axis_name="s", num_cores=NC, num_subcores=NS
)
_params = pltpu.CompilerParams(needs_layout_passes=False)


def _bin_of(x):
    t = (255.0 - x * 255.0).astype(jnp.int32)
    return jnp.maximum((LEVEL - 1) - t, 0)


@functools.partial(
    pl.kernel,
    out_type=jax.ShapeDtypeStruct((NW * HBINS,), jnp.float32),
    mesh=_mesh,
    scratch_types=[
        pltpu.VMEM((CH,), jnp.float32),
        pltpu.VMEM((CH,), jnp.float32),
        pltpu.VMEM((L * HBINS,), jnp.float32),
        pltpu.VMEM((HBINS,), jnp.float32),
        pltpu.SemaphoreType.DMA,
        pltpu.SemaphoreType.DMA,
    ],
    compiler_params=_params,
)
def _hist_kernel(x_hbm, part_hbm, buf0_v, buf1_v, hist_v, row_v, sem0, sem1):
    wid = lax.axis_index("c") * NS + lax.axis_index("s")
    base = wid * PER_W
    bufs, sems = (buf0_v, buf1_v), (sem0, sem1)
    lane = lax.iota(jnp.int32, L)
    ones = jnp.full((L,), 1.0, dtype=jnp.float32)

    def start_in(ci, b):
        pltpu.async_copy(x_hbm.at[pl.ds(base + ci * CH, CH)], bufs[b], sems[b])

    def wait_in(b):
        pltpu.make_async_copy(
            x_hbm.at[pl.ds(base, CH)], bufs[b], sems[b]
        ).wait()

    start_in(0, 0)
    start_in(1, 1)

    def zero_body(i, _):
        hist_v[pl.ds(i * L, L)] = jnp.zeros((L,), jnp.float32)
        return 0
    lax.fori_loop(0, (L * HBINS) // L, zero_body, 0)

    def chunk_pair(g, _):
        for b in range(2):
            ci = g * 2 + b
            wait_in(b)

            def vreg_body(vi, _, buf=bufs[b]):
                for u in range(UNROLL):
                    j = (vi * UNROLL + u) * L
                    idx = _bin_of(buf[pl.ds(j, L)]) * L + lane
                    plsc.addupdate_scatter(hist_v, [idx], ones)
                return 0
            lax.fori_loop(0, VPC // UNROLL, vreg_body, 0)

            @pl.when(ci + 2 < NCHUNK)
            def _():
                start_in(ci + 2, b)
        return 0
    lax.fori_loop(0, NCHUNK // 2, chunk_pair, 0)

    for k in range(HBINS // L):
        base_idx = (lane + k * L) * L
        acc = plsc.load_gather(hist_v, [base_idx])
        for l in range(1, L):
            acc = acc + plsc.load_gather(hist_v, [base_idx + l])
        row_v[pl.ds(k * L, L)] = acc

    pltpu.sync_copy(row_v, part_hbm.at[pl.ds(wid * HBINS, HBINS)])


@functools.partial(
    pl.kernel,
    out_type=jax.ShapeDtypeStruct((TOTAL,), jnp.float32),
    mesh=_mesh,
    scratch_types=[
        pltpu.VMEM((CH,), jnp.float32),
        pltpu.VMEM((CH,), jnp.float32),
        pltpu.VMEM((CH,), jnp.float32),
        pltpu.VMEM((CH,), jnp.float32),
        pltpu.VMEM((HBINS,), jnp.float32),
        pltpu.VMEM((HBINS,), jnp.float32),
        pltpu.VMEM((HBINS,), jnp.float32),
        pltpu.SemaphoreType.DMA,
        pltpu.SemaphoreType.DMA,
        pltpu.SemaphoreType.DMA,
        pltpu.SemaphoreType.DMA,
    ],
    compiler_params=_params,
)
def _map_kernel(x_hbm, part_hbm, out_hbm, buf0_v, buf1_v, obuf0_v, obuf1_v,
                p0_v, p1_v, cdf_v, isem0, isem1, osem0, osem1):
    wid = lax.axis_index("c") * NS + lax.axis_index("s")
    img = wid // 2
    base = wid * PER_W
    bufs, isems = (buf0_v, buf1_v), (isem0, isem1)
    obufs, osems = (obuf0_v, obuf1_v), (osem0, osem1)

    def start_in(ci, b):
        pltpu.async_copy(x_hbm.at[pl.ds(base + ci * CH, CH)], bufs[b], isems[b])

    def wait_in(b):
        pltpu.make_async_copy(
            x_hbm.at[pl.ds(base, CH)], bufs[b], isems[b]
        ).wait()

    def start_out(ci, b):
        pltpu.async_copy(
            obufs[b], out_hbm.at[pl.ds(base + ci * CH, CH)], osems[b]
        )

    def wait_out(b):
        pltpu.make_async_copy(
            obufs[b], out_hbm.at[pl.ds(base, CH)], osems[b]
        ).wait()

    start_in(0, 0)
    start_in(1, 1)

    pltpu.sync_copy(part_hbm.at[pl.ds((2 * img) * HBINS, HBINS)], p0_v)
    pltpu.sync_copy(part_hbm.at[pl.ds((2 * img + 1) * HBINS, HBINS)], p1_v)

    inv_n = jnp.float32(1.0 / NPIX)
    running = jnp.zeros((L,), jnp.float32)
    for k in range(HBINS // L):
        v = p0_v[pl.ds(k * L, L)] + p1_v[pl.ds(k * L, L)]
        c = plsc.cumsum(v) + running
        cdf_v[pl.ds(k * L, L)] = c * inv_n
        running = running + jnp.sum(v)

    def chunk_pair(g, _):
        for b in range(2):
            ci = g * 2 + b
            wait_in(b)

            @pl.when(ci >= 2)
            def _():
                wait_out(b)

            def vreg_body(vi, _, buf=bufs[b], obuf=obufs[b]):
                for u in range(UNROLL):
                    j = (vi * UNROLL + u) * L
                    bins = _bin_of(buf[pl.ds(j, L)])
                    obuf[pl.ds(j, L)] = plsc.load_gather(cdf_v, [bins])
                return 0
            lax.fori_loop(0, VPC // UNROLL, vreg_body, 0)

            start_out(ci, b)

            @pl.when(ci + 2 < NCHUNK)
            def _():
                start_in(ci + 2, b)
        return 0
    lax.fori_loop(0, NCHUNK // 2, chunk_pair, 0)

    wait_out(0)
    wait_out(1)


@jax.jit
def kernel(fusion):
    x = fusion.reshape(TOTAL)
    partials = _hist_kernel(x)
    out = _map_kernel(x, partials)
    return out.reshape(fusion.shape)


# speedup vs baseline: 6557.2462x; 2.7963x over previous
import functools

import jax
import jax.numpy as jnp
from jax import lax
from jax.experimental import pallas as pl
from jax.experimental.pallas import tpu as pltpu
from jax.experimental.pallas import tpu_sc as plsc

LEVEL = 255
NC, NS, L = 2, 16, 16
NW = NC * NS
B, C, H, W = 16, 3, 512, 512
NPIX = C * H * W
TOTAL = B * NPIX
PER_W = TOTAL // NW
CH = 16384
NCHUNK = PER_W // CH
VPC = CH // L
UNROLL = 16
HBINS = 256

_mesh = plsc.VectorSubcoreMesh(
    core_axis_name="c", subcore_axis_name="s", num_cores=NC, num_subcores=NS
)
_params = pltpu.CompilerParams(needs_layout_passes=False)


def _bin_of(x):
    t = (255.0 - x * 255.0).astype(jnp.int32)
    return jnp.maximum((LEVEL - 1) - t, 0)


@functools.partial(
    pl.kernel,
    out_type=jax.ShapeDtypeStruct((NW * HBINS,), jnp.float32),
    mesh=_mesh,
    scratch_types=[
        pltpu.VMEM((CH,), jnp.float32),
        pltpu.VMEM((CH,), jnp.float32),
        pltpu.VMEM((L * HBINS,), jnp.float32),
        pltpu.VMEM((HBINS,), jnp.float32),
        pltpu.SemaphoreType.DMA,
        pltpu.SemaphoreType.DMA,
    ],
    compiler_params=_params,
)
def _hist_kernel(x_hbm, part_hbm, buf0_v, buf1_v, hist_v, row_v, sem0, sem1):
    wid = lax.axis_index("c") * NS + lax.axis_index("s")
    base = wid * PER_W
    bufs, sems = (buf0_v, buf1_v), (sem0, sem1)
    lane = lax.iota(jnp.int32, L)
    ones = jnp.full((L,), 1.0, dtype=jnp.float32)

    def start_in(ci, b):
        pltpu.async_copy(x_hbm.at[pl.ds(base + ci * CH, CH)], bufs[b], sems[b])

    def wait_in(b):
        pltpu.make_async_copy(
            x_hbm.at[pl.ds(base, CH)], bufs[b], sems[b]
        ).wait()

    start_in(0, 0)
    start_in(1, 1)

    def zero_body(i, _):
        hist_v[pl.ds(i * L, L)] = jnp.zeros((L,), jnp.float32)
        return 0
    lax.fori_loop(0, (L * HBINS) // L, zero_body, 0)

    def chunk_pair(g, _):
        for b in range(2):
            ci = g * 2 + b
            wait_in(b)

            buf = bufs[b]

            @plsc.parallel_loop(0, VPC, unroll=UNROLL)
            def _(vi, buf=buf):
                j = vi * L
                idx = _bin_of(buf[pl.ds(j, L)]) * L + lane
                plsc.addupdate_scatter(hist_v, [idx], ones)

            @pl.when(ci + 2 < NCHUNK)
            def _():
                start_in(ci + 2, b)
        return 0
    lax.fori_loop(0, NCHUNK // 2, chunk_pair, 0)

    for k in range(HBINS // L):
        base_idx = (lane + k * L) * L
        acc = plsc.load_gather(hist_v, [base_idx])
        for l in range(1, L):
            acc = acc + plsc.load_gather(hist_v, [base_idx + l])
        row_v[pl.ds(k * L, L)] = acc

    pltpu.sync_copy(row_v, part_hbm.at[pl.ds(wid * HBINS, HBINS)])


@functools.partial(
    pl.kernel,
    out_type=jax.ShapeDtypeStruct((TOTAL,), jnp.float32),
    mesh=_mesh,
    scratch_types=[
        pltpu.VMEM((CH,), jnp.float32),
        pltpu.VMEM((CH,), jnp.float32),
        pltpu.VMEM((CH,), jnp.float32),
        pltpu.VMEM((CH,), jnp.float32),
        pltpu.VMEM((HBINS,), jnp.float32),
        pltpu.VMEM((HBINS,), jnp.float32),
        pltpu.VMEM((HBINS,), jnp.float32),
        pltpu.SemaphoreType.DMA,
        pltpu.SemaphoreType.DMA,
        pltpu.SemaphoreType.DMA,
        pltpu.SemaphoreType.DMA,
    ],
    compiler_params=_params,
)
def _map_kernel(x_hbm, part_hbm, out_hbm, buf0_v, buf1_v, obuf0_v, obuf1_v,
                p0_v, p1_v, cdf_v, isem0, isem1, osem0, osem1):
    wid = lax.axis_index("c") * NS + lax.axis_index("s")
    img = wid // 2
    base = wid * PER_W
    bufs, isems = (buf0_v, buf1_v), (isem0, isem1)
    obufs, osems = (obuf0_v, obuf1_v), (osem0, osem1)

    def start_in(ci, b):
        pltpu.async_copy(x_hbm.at[pl.ds(base + ci * CH, CH)], bufs[b], isems[b])

    def wait_in(b):
        pltpu.make_async_copy(
            x_hbm.at[pl.ds(base, CH)], bufs[b], isems[b]
        ).wait()

    def start_out(ci, b):
        pltpu.async_copy(
            obufs[b], out_hbm.at[pl.ds(base + ci * CH, CH)], osems[b]
        )

    def wait_out(b):
        pltpu.make_async_copy(
            obufs[b], out_hbm.at[pl.ds(base, CH)], osems[b]
        ).wait()

    start_in(0, 0)
    start_in(1, 1)

    pltpu.sync_copy(part_hbm.at[pl.ds((2 * img) * HBINS, HBINS)], p0_v)
    pltpu.sync_copy(part_hbm.at[pl.ds((2 * img + 1) * HBINS, HBINS)], p1_v)

    inv_n = jnp.float32(1.0 / NPIX)
    running = jnp.zeros((L,), jnp.float32)
    for k in range(HBINS // L):
        v = p0_v[pl.ds(k * L, L)] + p1_v[pl.ds(k * L, L)]
        c = plsc.cumsum(v) + running
        cdf_v[pl.ds(k * L, L)] = c * inv_n
        running = running + jnp.sum(v)

    def chunk_pair(g, _):
        for b in range(2):
            ci = g * 2 + b
            wait_in(b)

            @pl.when(ci >= 2)
            def _():
                wait_out(b)

            @plsc.parallel_loop(0, VPC, unroll=UNROLL)
            def _(vi, buf=bufs[b], obuf=obufs[b]):
                j = vi * L
                bins = _bin_of(buf[pl.ds(j, L)])
                obuf[pl.ds(j, L)] = plsc.load_gather(cdf_v, [bins])

            start_out(ci, b)

            @pl.when(ci + 2 < NCHUNK)
            def _():
                start_in(ci + 2, b)
        return 0
    lax.fori_loop(0, NCHUNK // 2, chunk_pair, 0)

    wait_out(0)
    wait_out(1)


@jax.jit
def kernel(fusion):
    x = fusion.reshape(TOTAL)
    partials = _hist_kernel(x)
    out = _map_kernel(x, partials)
    return out.reshape(fusion.shape)


# speedup vs baseline: 6679.6747x; 1.0187x over previous
import functools

import jax
import jax.numpy as jnp
from jax import lax
from jax.experimental import pallas as pl
from jax.experimental.pallas import tpu as pltpu
from jax.experimental.pallas import tpu_sc as plsc

LEVEL = 255
NC, NS, L = 2, 16, 16
NW = NC * NS
B, C, H, W = 16, 3, 512, 512
NPIX = C * H * W
TOTAL = B * NPIX
PER_W = TOTAL // NW
CH = 16384
NCHUNK = PER_W // CH
VPC = CH // L
UNROLL = 16
HBINS = 256

_mesh = plsc.VectorSubcoreMesh(
    core_axis_name="c", subcore_axis_name="s", num_cores=NC, num_subcores=NS
)
_params = pltpu.CompilerParams(needs_layout_passes=False)


def _bin_of(x):
    t = (255.0 - x * 255.0).astype(jnp.int32)
    return jnp.maximum((LEVEL - 1) - t, 0)


@functools.partial(
    pl.kernel,
    out_type=jax.ShapeDtypeStruct((TOTAL,), jnp.float32),
    mesh=_mesh,
    scratch_types=[
        pltpu.VMEM((CH,), jnp.float32),
        pltpu.VMEM((CH,), jnp.float32),
        pltpu.VMEM((CH,), jnp.float32),
        pltpu.VMEM((CH,), jnp.float32),
        pltpu.VMEM((L * HBINS,), jnp.float32),
        pltpu.VMEM((HBINS,), jnp.float32),
        pltpu.VMEM((HBINS,), jnp.float32),
        pltpu.VMEM((HBINS,), jnp.float32),
        pltpu.VMEM_SHARED((NS, HBINS), jnp.float32),
        pltpu.SemaphoreType.DMA,
        pltpu.SemaphoreType.DMA,
        pltpu.SemaphoreType.DMA,
        pltpu.SemaphoreType.DMA,
    ],
    compiler_params=_params,
)
def _equalize_kernel(x_hbm, out_hbm, buf0_v, buf1_v, obuf0_v, obuf1_v,
                     hist_v, row_v, prow_v, cdf_v, shared_sp,
                     isem0, isem1, osem0, osem1):
    s = lax.axis_index("s")
    img = lax.axis_index("c") * (NS // 2) + s // 2
    base = img * NPIX + (s % 2) * PER_W
    bufs, isems = (buf0_v, buf1_v), (isem0, isem1)
    obufs, osems = (obuf0_v, obuf1_v), (osem0, osem1)
    lane = lax.iota(jnp.int32, L)
    ones = jnp.full((L,), 1.0, dtype=jnp.float32)

    def start_in(ci, b):
        pltpu.async_copy(x_hbm.at[pl.ds(base + ci * CH, CH)], bufs[b], isems[b])

    def wait_in(b):
        pltpu.make_async_copy(
            x_hbm.at[pl.ds(base, CH)], bufs[b], isems[b]
        ).wait()

    def start_out(ci, b):
        pltpu.async_copy(
            obufs[b], out_hbm.at[pl.ds(base + ci * CH, CH)], osems[b]
        )

    def wait_out(b):
        pltpu.make_async_copy(
            obufs[b], out_hbm.at[pl.ds(base, CH)], osems[b]
        ).wait()

    start_in(0, 0)
    start_in(1, 1)

    @plsc.parallel_loop(0, (L * HBINS) // L)
    def _(i):
        hist_v[pl.ds(i * L, L)] = jnp.zeros((L,), jnp.float32)

    def hist_pair(g, _):
        for b in range(2):
            ci = g * 2 + b
            wait_in(b)

            @plsc.parallel_loop(0, VPC, unroll=UNROLL)
            def _(vi, buf=bufs[b]):
                idx = _bin_of(buf[pl.ds(vi * L, L)]) * L + lane
                plsc.addupdate_scatter(hist_v, [idx], ones)

            @pl.when(ci + 2 < NCHUNK)
            def _():
                start_in(ci + 2, b)
        return 0
    lax.fori_loop(0, NCHUNK // 2, hist_pair, 0)

    for k in range(HBINS // L):
        base_idx = (lane + k * L) * L
        acc = plsc.load_gather(hist_v, [base_idx])
        for l in range(1, L):
            acc = acc + plsc.load_gather(hist_v, [base_idx + l])
        row_v[pl.ds(k * L, L)] = acc

    pltpu.sync_copy(row_v, shared_sp.at[s])

    start_in(0, 0)
    start_in(1, 1)

    plsc.subcore_barrier()
    pltpu.sync_copy(shared_sp.at[s ^ 1], prow_v)

    inv_n = jnp.float32(1.0 / NPIX)
    running = jnp.zeros((L,), jnp.float32)
    for k in range(HBINS // L):
        v = row_v[pl.ds(k * L, L)] + prow_v[pl.ds(k * L, L)]
        c = plsc.cumsum(v) + running
        cdf_v[pl.ds(k * L, L)] = c * inv_n
        running = running + jnp.sum(v)

    def map_pair(g, _):
        for b in range(2):
            ci = g * 2 + b
            wait_in(b)

            @pl.when(ci >= 2)
            def _():
                wait_out(b)

            @plsc.parallel_loop(0, VPC, unroll=UNROLL)
            def _(vi, buf=bufs[b], obuf=obufs[b]):
                j = vi * L
                bins = _bin_of(buf[pl.ds(j, L)])
                obuf[pl.ds(j, L)] = plsc.load_gather(cdf_v, [bins])

            start_out(ci, b)

            @pl.when(ci + 2 < NCHUNK)
            def _():
                start_in(ci + 2, b)
        return 0
    lax.fori_loop(0, NCHUNK // 2, map_pair, 0)

    wait_out(0)
    wait_out(1)


@jax.jit
def kernel(fusion):
    x = fusion.reshape(TOTAL)
    out = _equalize_kernel(x)
    return out.reshape(fusion.shape)
